# baseline (speedup 1.0000x reference)
import functools

import jax
import jax.numpy as jnp
from jax import lax
from jax.experimental import pallas as pl
from jax.experimental.pallas import tpu as pltpu
from jax.experimental.pallas import tpu_sc as plsc

T, D, E, H = 2048, 768, 8, 3072
K = 2
P = T * K
M = 256
NB = P // M + E
RS = NB * M
_NEG = jnp.finfo(jnp.float32).min

_NC, _NS = 2, 16
_NW = _NC * _NS
_PPW = P // _NW
_TPW = T // _NW
_TT = 32


def _gate_kernel(x_ref, gw_ref, logits_ref, pos_ref, w_ref, nb_ref):
    x = x_ref[...].astype(jnp.bfloat16)
    gw = gw_ref[...].astype(jnp.bfloat16)
    logits = lax.dot_general(
        x, gw, (((1,), (1,)), ((), ())),
        preferred_element_type=jnp.float32)
    logits_ref[...] = logits
    eidx = lax.broadcasted_iota(jnp.int32, (T, E), 1)
    m1 = jnp.max(logits, axis=1, keepdims=True)
    i1 = jnp.min(jnp.where(logits == m1, eidx, E), axis=1, keepdims=True)
    masked = jnp.where(eidx == i1, _NEG, logits)
    m2 = jnp.max(masked, axis=1, keepdims=True)
    i2 = jnp.min(jnp.where(masked == m2, eidx, E), axis=1, keepdims=True)
    ex = jnp.exp(m2 - m1)
    denom = 1.0 + ex
    w_ref[...] = jnp.concatenate([1.0 / denom, ex / denom], axis=1)

    oh1 = (eidx == i1).astype(jnp.bfloat16)
    oh2 = (eidx == i2).astype(jnp.bfloat16)
    ohb = jnp.concatenate([oh1, oh2], axis=1)
    rI = lax.broadcasted_iota(jnp.int32, (T, T), 0)
    cI = lax.broadcasted_iota(jnp.int32, (T, T), 1)
    ltri = (cI <= rI).astype(jnp.bfloat16)
    cum = lax.dot_general(
        ltri, ohb, (((1,), (0,)), ((), ())),
        preferred_element_type=jnp.float32)
    c0 = cum[T - 1:T, 0:E]
    counts = c0 + cum[T - 1:T, E:2 * E]
    nb = jnp.floor((counts + (M - 1)) * (1.0 / M))
    nb_cum = lax.dot_general(
        nb.astype(jnp.bfloat16),
        (lax.broadcasted_iota(jnp.int32, (E, E), 0)
         <= lax.broadcasted_iota(jnp.int32, (E, E), 1)).astype(jnp.bfloat16),
        (((1,), (0,)), ((), ())),
        preferred_element_type=jnp.float32)
    pbase = (nb_cum - nb) * float(M)
    o1f = oh1.astype(jnp.float32)
    o2f = oh2.astype(jnp.float32)
    pos0 = jnp.sum(o1f * (cum[:, 0:E] + pbase), axis=1, keepdims=True) - 1.0
    pos1 = jnp.sum(o2f * (cum[:, E:2 * E] + c0 + pbase),
                   axis=1, keepdims=True) - 1.0
    pos_ref[...] = jnp.concatenate([pos0, pos1], axis=1).astype(jnp.int32)
    nb_ref[...] = jnp.broadcast_to(nb, (E, E)).astype(jnp.int32)


def _gate(x2, gate_w):
    return pl.pallas_call(
        _gate_kernel,
        out_shape=(jax.ShapeDtypeStruct((T, E), jnp.float32),
                   jax.ShapeDtypeStruct((T, K), jnp.int32),
                   jax.ShapeDtypeStruct((T, K), jnp.float32),
                   jax.ShapeDtypeStruct((E, E), jnp.int32)),
    )(x2, gate_w)


def _blocktable(nbrow):
    nb_cum = jnp.cumsum(nbrow)
    total_nb = nb_cum[E - 1]
    bidx = jnp.arange(NB, dtype=jnp.int32)
    be = jnp.sum((bidx[:, None] >= nb_cum[None, :]).astype(jnp.int32), axis=1)
    last_be = jnp.clip(
        jnp.sum((total_nb - 1 >= nb_cum).astype(jnp.int32)), 0, E - 1)
    be = jnp.where(bidx < total_nb, jnp.clip(be, 0, E - 1), last_be)
    act = (bidx < total_nb).astype(jnp.int32)
    return be.astype(jnp.int32), act


def _dispatch_body(x_hbm, pos_hbm, ww_hbm, xs_hbm, sw_hbm,
                   idx_v, pos_v, rows_v, w_v, sem):
    c = lax.axis_index("c")
    s = lax.axis_index("s")
    wid = s * _NC + c
    base = wid * _PPW
    pltpu.sync_copy(pos_hbm.at[pl.ds(base, _PPW)], pos_v)
    pltpu.sync_copy(ww_hbm.at[pl.ds(base, _PPW)], w_v)
    for j in range(_PPW // 16):
        idx_v[pl.ds(j * 16, 16)] = lax.bitwise_and(
            lax.broadcasted_iota(jnp.int32, (16,), 0) + (base + j * 16),
            T - 1)
    pltpu.async_copy(x_hbm.at[idx_v], rows_v, sem).wait()
    pltpu.sync_copy(rows_v, xs_hbm.at[pos_v])
    pltpu.sync_copy(w_v, sw_hbm.at[pos_v])


def _dispatch(x2, pos, ww):
    mesh = plsc.VectorSubcoreMesh(core_axis_name="c", subcore_axis_name="s")
    f = pl.kernel(
        _dispatch_body,
        mesh=mesh,
        out_type=(jax.ShapeDtypeStruct((RS, D), jnp.float32),
                  jax.ShapeDtypeStruct((RS, 128), jnp.float32)),
        scratch_types=[
            pltpu.VMEM((_PPW,), jnp.int32),
            pltpu.VMEM((_PPW,), jnp.int32),
            pltpu.VMEM((_PPW, D), jnp.float32),
            pltpu.VMEM((_PPW, 128), jnp.float32),
            pltpu.SemaphoreType.DMA,
        ],
    )
    return f(x2, pos, ww)


def _erf(z):
    return lax.erf(z)


def _gelu_exact(h):
    return 0.5 * h * (1.0 + _erf(h * 0.7071067811865476))


def _ffn_kernel(be_ref, act_ref, xs_ref, sw_ref, w1_ref, w2_ref, y_ref,
                w1b_ref, w2b_ref):
    b = pl.program_id(0)

    @pl.when(act_ref[b] == 1)
    def _():
        prev = be_ref[jnp.maximum(b - 1, 0)]

        @pl.when((b == 0) | (be_ref[b] != prev))
        def _cast():
            w1b_ref[...] = w1_ref[0].astype(jnp.bfloat16)
            w2b_ref[...] = w2_ref[0].astype(jnp.bfloat16)

        xb = xs_ref[...].astype(jnp.bfloat16)
        h = lax.dot_general(
            xb, w1b_ref[...], (((1,), (1,)), ((), ())),
            preferred_element_type=jnp.float32)
        a = _gelu_exact(h).astype(jnp.bfloat16)
        y = lax.dot_general(
            a, w2b_ref[...], (((1,), (1,)), ((), ())),
            preferred_element_type=jnp.float32)
        y_ref[...] = y * sw_ref[:, 0:1]


def _ffn(xs, sw, W1, W2, be, act):
    grid_spec = pltpu.PrefetchScalarGridSpec(
        num_scalar_prefetch=2,
        grid=(NB,),
        in_specs=[
            pl.BlockSpec((M, D), lambda b, be, act: (b, 0)),
            pl.BlockSpec((M, 128), lambda b, be, act: (b, 0)),
            pl.BlockSpec((1, H, D), lambda b, be, act: (be[b], 0, 0)),
            pl.BlockSpec((1, D, H), lambda b, be, act: (be[b], 0, 0)),
        ],
        out_specs=pl.BlockSpec((M, D), lambda b, be, act: (b, 0)),
        scratch_shapes=[pltpu.VMEM((H, D), jnp.bfloat16),
                        pltpu.VMEM((D, H), jnp.bfloat16)],
    )
    return pl.pallas_call(
        _ffn_kernel,
        grid_spec=grid_spec,
        out_shape=jax.ShapeDtypeStruct((RS, D), jnp.float32),
    )(be, act, xs, sw, W1, W2)


def _combine_body(y_hbm, pos_hbm, out_hbm,
                  posa0_v, posa1_v, posb0_v, posb1_v,
                  rowsa0_v, rowsa1_v, rowsb0_v, rowsb1_v,
                  acc_v, sema, semb):
    c = lax.axis_index("c")
    s = lax.axis_index("s")
    wid = s * _NC + c
    tbase = wid * _TPW
    pltpu.sync_copy(pos_hbm.at[pl.ds(tbase, _TT)], posa0_v)
    pltpu.sync_copy(pos_hbm.at[pl.ds(T + tbase, _TT)], posa1_v)
    pltpu.sync_copy(pos_hbm.at[pl.ds(tbase + _TT, _TT)], posb0_v)
    pltpu.sync_copy(pos_hbm.at[pl.ds(T + tbase + _TT, _TT)], posb1_v)
    cpa0 = pltpu.async_copy(y_hbm.at[posa0_v], rowsa0_v, sema)
    cpa1 = pltpu.async_copy(y_hbm.at[posa1_v], rowsa1_v, sema)
    cpb0 = pltpu.async_copy(y_hbm.at[posb0_v], rowsb0_v, semb)
    cpb1 = pltpu.async_copy(y_hbm.at[posb1_v], rowsb1_v, semb)

    def mk_body(r0_v, r1_v):
        def body(t, _):
            for j in range(D // 16):
                sl = pl.ds(j * 16, 16)
                acc_v[t, sl] = r0_v[t, sl] + r1_v[t, sl]
            return 0
        return body

    cpa0.wait()
    cpa1.wait()
    lax.fori_loop(0, _TT, mk_body(rowsa0_v, rowsa1_v), 0)
    pltpu.sync_copy(acc_v, out_hbm.at[pl.ds(tbase, _TT)])
    cpb0.wait()
    cpb1.wait()
    lax.fori_loop(0, _TT, mk_body(rowsb0_v, rowsb1_v), 0)
    pltpu.sync_copy(acc_v, out_hbm.at[pl.ds(tbase + _TT, _TT)])


def _combine(y, pos):
    mesh = plsc.VectorSubcoreMesh(core_axis_name="c", subcore_axis_name="s")
    f = pl.kernel(
        _combine_body,
        mesh=mesh,
        out_type=jax.ShapeDtypeStruct((T, D), jnp.float32),
        scratch_types=[
            pltpu.VMEM((_TT,), jnp.int32),
            pltpu.VMEM((_TT,), jnp.int32),
            pltpu.VMEM((_TT,), jnp.int32),
            pltpu.VMEM((_TT,), jnp.int32),
            pltpu.VMEM((_TT, D), jnp.float32),
            pltpu.VMEM((_TT, D), jnp.float32),
            pltpu.VMEM((_TT, D), jnp.float32),
            pltpu.VMEM((_TT, D), jnp.float32),
            pltpu.VMEM((_TT, D), jnp.float32),
            pltpu.SemaphoreType.DMA,
            pltpu.SemaphoreType.DMA,
        ],
    )
    return f(y, pos)


@jax.jit
def kernel(x, gate_w, W1, W2):
    b, s, d = x.shape
    x2 = x.reshape(s, d)
    logits, pos2, w, nbm = _gate(x2, gate_w)
    pos = pos2.T.reshape(P)
    be, act = _blocktable(nbm[0])
    ww = jnp.broadcast_to(w.T.reshape(P, 1), (P, 128))
    xs, sw = _dispatch(x2, pos, ww)
    y = _ffn(xs, sw, W1, W2, be, act)
    out = _combine(y, pos)
    return out.reshape(b, s, d), logits.reshape(b, s, E)


# speedup vs baseline: 1.0750x; 1.0750x over previous
import functools

import jax
import jax.numpy as jnp
from jax import lax
from jax.experimental import pallas as pl
from jax.experimental.pallas import tpu as pltpu
from jax.experimental.pallas import tpu_sc as plsc

T, D, E, H = 2048, 768, 8, 3072
K = 2
P = T * K
M = 512
NB = P // M + E
RS = NB * M
_NEG = jnp.finfo(jnp.float32).min

_NC, _NS = 2, 16
_NW = _NC * _NS
_PPW = P // _NW
_TPW = T // _NW
_TT = 32


def _gate_kernel(x_ref, gw_ref, logits_ref, pos_ref, w_ref, nb_ref):
    x = x_ref[...].astype(jnp.bfloat16)
    gw = gw_ref[...].astype(jnp.bfloat16)
    logits = lax.dot_general(
        x, gw, (((1,), (1,)), ((), ())),
        preferred_element_type=jnp.float32)
    logits_ref[...] = logits
    eidx = lax.broadcasted_iota(jnp.int32, (T, E), 1)
    m1 = jnp.max(logits, axis=1, keepdims=True)
    i1 = jnp.min(jnp.where(logits == m1, eidx, E), axis=1, keepdims=True)
    masked = jnp.where(eidx == i1, _NEG, logits)
    m2 = jnp.max(masked, axis=1, keepdims=True)
    i2 = jnp.min(jnp.where(masked == m2, eidx, E), axis=1, keepdims=True)
    ex = jnp.exp(m2 - m1)
    denom = 1.0 + ex
    w_ref[...] = jnp.concatenate([1.0 / denom, ex / denom], axis=1)

    oh1 = (eidx == i1).astype(jnp.bfloat16)
    oh2 = (eidx == i2).astype(jnp.bfloat16)
    ohb = jnp.concatenate([oh1, oh2], axis=1)
    rI = lax.broadcasted_iota(jnp.int32, (T, T), 0)
    cI = lax.broadcasted_iota(jnp.int32, (T, T), 1)
    ltri = (cI <= rI).astype(jnp.bfloat16)
    cum = lax.dot_general(
        ltri, ohb, (((1,), (0,)), ((), ())),
        preferred_element_type=jnp.float32)
    c0 = cum[T - 1:T, 0:E]
    counts = c0 + cum[T - 1:T, E:2 * E]
    nb = jnp.floor((counts + (M - 1)) * (1.0 / M))
    nb_cum = lax.dot_general(
        nb.astype(jnp.bfloat16),
        (lax.broadcasted_iota(jnp.int32, (E, E), 0)
         <= lax.broadcasted_iota(jnp.int32, (E, E), 1)).astype(jnp.bfloat16),
        (((1,), (0,)), ((), ())),
        preferred_element_type=jnp.float32)
    pbase = (nb_cum - nb) * float(M)
    o1f = oh1.astype(jnp.float32)
    o2f = oh2.astype(jnp.float32)
    pos0 = jnp.sum(o1f * (cum[:, 0:E] + pbase), axis=1, keepdims=True) - 1.0
    pos1 = jnp.sum(o2f * (cum[:, E:2 * E] + c0 + pbase),
                   axis=1, keepdims=True) - 1.0
    pos_ref[...] = jnp.concatenate([pos0, pos1], axis=1).astype(jnp.int32)
    nb_ref[...] = jnp.broadcast_to(nb, (E, E)).astype(jnp.int32)


def _gate(x2, gate_w):
    return pl.pallas_call(
        _gate_kernel,
        out_shape=(jax.ShapeDtypeStruct((T, E), jnp.float32),
                   jax.ShapeDtypeStruct((T, K), jnp.int32),
                   jax.ShapeDtypeStruct((T, K), jnp.float32),
                   jax.ShapeDtypeStruct((E, E), jnp.int32)),
    )(x2, gate_w)


def _blocktable(nbrow):
    nb_cum = jnp.cumsum(nbrow)
    total_nb = nb_cum[E - 1]
    bidx = jnp.arange(NB, dtype=jnp.int32)
    be = jnp.sum((bidx[:, None] >= nb_cum[None, :]).astype(jnp.int32), axis=1)
    last_be = jnp.clip(
        jnp.sum((total_nb - 1 >= nb_cum).astype(jnp.int32)), 0, E - 1)
    be = jnp.where(bidx < total_nb, jnp.clip(be, 0, E - 1), last_be)
    act = (bidx < total_nb).astype(jnp.int32)
    return be.astype(jnp.int32), act


def _dispatch_body(x_hbm, pos_hbm, ww_hbm, xs_hbm, sw_hbm,
                   idx_v, pos_v, rows_v, w_v, sem):
    c = lax.axis_index("c")
    s = lax.axis_index("s")
    wid = s * _NC + c
    base = wid * _PPW
    pltpu.sync_copy(pos_hbm.at[pl.ds(base, _PPW)], pos_v)
    pltpu.sync_copy(ww_hbm.at[pl.ds(base, _PPW)], w_v)
    for j in range(_PPW // 16):
        idx_v[pl.ds(j * 16, 16)] = lax.bitwise_and(
            lax.broadcasted_iota(jnp.int32, (16,), 0) + (base + j * 16),
            T - 1)
    pltpu.async_copy(x_hbm.at[idx_v], rows_v, sem).wait()
    pltpu.sync_copy(rows_v, xs_hbm.at[pos_v])
    pltpu.sync_copy(w_v, sw_hbm.at[pos_v])


def _dispatch(x2, pos, ww):
    mesh = plsc.VectorSubcoreMesh(core_axis_name="c", subcore_axis_name="s")
    f = pl.kernel(
        _dispatch_body,
        mesh=mesh,
        out_type=(jax.ShapeDtypeStruct((RS, D), jnp.float32),
                  jax.ShapeDtypeStruct((RS, 128), jnp.float32)),
        scratch_types=[
            pltpu.VMEM((_PPW,), jnp.int32),
            pltpu.VMEM((_PPW,), jnp.int32),
            pltpu.VMEM((_PPW, D), jnp.float32),
            pltpu.VMEM((_PPW, 128), jnp.float32),
            pltpu.SemaphoreType.DMA,
        ],
    )
    return f(x2, pos, ww)


def _erf(z):
    return lax.erf(z)


def _gelu_exact(h):
    return 0.5 * h * (1.0 + _erf(h * 0.7071067811865476))


def _ffn_kernel(be_ref, act_ref, xs_ref, sw_ref, w1_ref, w2_ref, y_ref,
                w1b_ref, w2b_ref):
    b = pl.program_id(0)

    @pl.when(act_ref[b] == 1)
    def _():
        prev = be_ref[jnp.maximum(b - 1, 0)]

        @pl.when((b == 0) | (be_ref[b] != prev))
        def _cast():
            w1b_ref[...] = w1_ref[0].astype(jnp.bfloat16)
            w2b_ref[...] = w2_ref[0].astype(jnp.bfloat16)

        xb = xs_ref[...].astype(jnp.bfloat16)
        h = lax.dot_general(
            xb, w1b_ref[...], (((1,), (1,)), ((), ())),
            preferred_element_type=jnp.float32)
        a = _gelu_exact(h).astype(jnp.bfloat16)
        y = lax.dot_general(
            a, w2b_ref[...], (((1,), (1,)), ((), ())),
            preferred_element_type=jnp.float32)
        y_ref[...] = y * sw_ref[:, 0:1]


def _ffn(xs, sw, W1, W2, be, act):
    grid_spec = pltpu.PrefetchScalarGridSpec(
        num_scalar_prefetch=2,
        grid=(NB,),
        in_specs=[
            pl.BlockSpec((M, D), lambda b, be, act: (b, 0)),
            pl.BlockSpec((M, 128), lambda b, be, act: (b, 0)),
            pl.BlockSpec((1, H, D), lambda b, be, act: (be[b], 0, 0)),
            pl.BlockSpec((1, D, H), lambda b, be, act: (be[b], 0, 0)),
        ],
        out_specs=pl.BlockSpec((M, D), lambda b, be, act: (b, 0)),
        scratch_shapes=[pltpu.VMEM((H, D), jnp.bfloat16),
                        pltpu.VMEM((D, H), jnp.bfloat16)],
    )
    return pl.pallas_call(
        _ffn_kernel,
        grid_spec=grid_spec,
        out_shape=jax.ShapeDtypeStruct((RS, D), jnp.float32),
    )(be, act, xs, sw, W1, W2)


def _combine_body(y_hbm, pos_hbm, out_hbm,
                  posa0_v, posa1_v, posb0_v, posb1_v,
                  rowsa0_v, rowsa1_v, rowsb0_v, rowsb1_v,
                  acc_v, sema, semb):
    c = lax.axis_index("c")
    s = lax.axis_index("s")
    wid = s * _NC + c
    tbase = wid * _TPW
    pltpu.sync_copy(pos_hbm.at[pl.ds(tbase, _TT)], posa0_v)
    pltpu.sync_copy(pos_hbm.at[pl.ds(T + tbase, _TT)], posa1_v)
    pltpu.sync_copy(pos_hbm.at[pl.ds(tbase + _TT, _TT)], posb0_v)
    pltpu.sync_copy(pos_hbm.at[pl.ds(T + tbase + _TT, _TT)], posb1_v)
    cpa0 = pltpu.async_copy(y_hbm.at[posa0_v], rowsa0_v, sema)
    cpa1 = pltpu.async_copy(y_hbm.at[posa1_v], rowsa1_v, sema)
    cpb0 = pltpu.async_copy(y_hbm.at[posb0_v], rowsb0_v, semb)
    cpb1 = pltpu.async_copy(y_hbm.at[posb1_v], rowsb1_v, semb)

    def mk_body(r0_v, r1_v):
        def body(t, _):
            for j in range(D // 16):
                sl = pl.ds(j * 16, 16)
                acc_v[t, sl] = r0_v[t, sl] + r1_v[t, sl]
            return 0
        return body

    cpa0.wait()
    cpa1.wait()
    lax.fori_loop(0, _TT, mk_body(rowsa0_v, rowsa1_v), 0)
    pltpu.sync_copy(acc_v, out_hbm.at[pl.ds(tbase, _TT)])
    cpb0.wait()
    cpb1.wait()
    lax.fori_loop(0, _TT, mk_body(rowsb0_v, rowsb1_v), 0)
    pltpu.sync_copy(acc_v, out_hbm.at[pl.ds(tbase + _TT, _TT)])


def _combine(y, pos):
    mesh = plsc.VectorSubcoreMesh(core_axis_name="c", subcore_axis_name="s")
    f = pl.kernel(
        _combine_body,
        mesh=mesh,
        out_type=jax.ShapeDtypeStruct((T, D), jnp.float32),
        scratch_types=[
            pltpu.VMEM((_TT,), jnp.int32),
            pltpu.VMEM((_TT,), jnp.int32),
            pltpu.VMEM((_TT,), jnp.int32),
            pltpu.VMEM((_TT,), jnp.int32),
            pltpu.VMEM((_TT, D), jnp.float32),
            pltpu.VMEM((_TT, D), jnp.float32),
            pltpu.VMEM((_TT, D), jnp.float32),
            pltpu.VMEM((_TT, D), jnp.float32),
            pltpu.VMEM((_TT, D), jnp.float32),
            pltpu.SemaphoreType.DMA,
            pltpu.SemaphoreType.DMA,
        ],
    )
    return f(y, pos)


@jax.jit
def kernel(x, gate_w, W1, W2):
    b, s, d = x.shape
    x2 = x.reshape(s, d)
    logits, pos2, w, nbm = _gate(x2, gate_w)
    pos = pos2.T.reshape(P)
    be, act = _blocktable(nbm[0])
    ww = jnp.broadcast_to(w.T.reshape(P, 1), (P, 128))
    xs, sw = _dispatch(x2, pos, ww)
    y = _ffn(xs, sw, W1, W2, be, act)
    out = _combine(y, pos)
    return out.reshape(b, s, d), logits.reshape(b, s, E)


# speedup vs baseline: 1.0878x; 1.0119x over previous
import functools

import jax
import jax.numpy as jnp
from jax import lax
from jax.experimental import pallas as pl
from jax.experimental.pallas import tpu as pltpu
from jax.experimental.pallas import tpu_sc as plsc

T, D, E, H = 2048, 768, 8, 3072
K = 2
P = T * K
M = 512
NB = P // M + E
RS = NB * M
_NEG = jnp.finfo(jnp.float32).min

_NC, _NS = 2, 16
_NW = _NC * _NS
_PPW = P // _NW
_TPW = T // _NW
_TT = 32


def _gate_kernel(x_ref, gw_ref, logits_ref, pos_ref, w_ref, nb_ref):
    x = x_ref[...].astype(jnp.bfloat16)
    gw = gw_ref[...].astype(jnp.bfloat16)
    logits = lax.dot_general(
        x, gw, (((1,), (1,)), ((), ())),
        preferred_element_type=jnp.float32)
    logits_ref[...] = logits
    eidx = lax.broadcasted_iota(jnp.int32, (T, E), 1)
    m1 = jnp.max(logits, axis=1, keepdims=True)
    i1 = jnp.min(jnp.where(logits == m1, eidx, E), axis=1, keepdims=True)
    masked = jnp.where(eidx == i1, _NEG, logits)
    m2 = jnp.max(masked, axis=1, keepdims=True)
    i2 = jnp.min(jnp.where(masked == m2, eidx, E), axis=1, keepdims=True)
    ex = jnp.exp(m2 - m1)
    denom = 1.0 + ex
    w_ref[...] = jnp.concatenate([1.0 / denom, ex / denom], axis=1)

    oh1 = (eidx == i1).astype(jnp.bfloat16)
    oh2 = (eidx == i2).astype(jnp.bfloat16)
    ohb = jnp.concatenate([oh1, oh2], axis=1)
    rI = lax.broadcasted_iota(jnp.int32, (T, T), 0)
    cI = lax.broadcasted_iota(jnp.int32, (T, T), 1)
    ltri = (cI <= rI).astype(jnp.bfloat16)
    cum = lax.dot_general(
        ltri, ohb, (((1,), (0,)), ((), ())),
        preferred_element_type=jnp.float32)
    c0 = cum[T - 1:T, 0:E]
    counts = c0 + cum[T - 1:T, E:2 * E]
    nb = jnp.floor((counts + (M - 1)) * (1.0 / M))
    nb_cum = lax.dot_general(
        nb.astype(jnp.bfloat16),
        (lax.broadcasted_iota(jnp.int32, (E, E), 0)
         <= lax.broadcasted_iota(jnp.int32, (E, E), 1)).astype(jnp.bfloat16),
        (((1,), (0,)), ((), ())),
        preferred_element_type=jnp.float32)
    pbase = (nb_cum - nb) * float(M)
    o1f = oh1.astype(jnp.float32)
    o2f = oh2.astype(jnp.float32)
    pos0 = jnp.sum(o1f * (cum[:, 0:E] + pbase), axis=1, keepdims=True) - 1.0
    pos1 = jnp.sum(o2f * (cum[:, E:2 * E] + c0 + pbase),
                   axis=1, keepdims=True) - 1.0
    pos_ref[...] = jnp.concatenate([pos0, pos1], axis=1).astype(jnp.int32)
    nb_ref[...] = jnp.broadcast_to(nb, (E, E)).astype(jnp.int32)


def _gate(x2, gate_w):
    return pl.pallas_call(
        _gate_kernel,
        out_shape=(jax.ShapeDtypeStruct((T, E), jnp.float32),
                   jax.ShapeDtypeStruct((T, K), jnp.int32),
                   jax.ShapeDtypeStruct((T, K), jnp.float32),
                   jax.ShapeDtypeStruct((E, E), jnp.int32)),
    )(x2, gate_w)


def _blocktable(nbrow):
    nb_cum = jnp.cumsum(nbrow)
    total_nb = nb_cum[E - 1]
    bidx = jnp.arange(NB, dtype=jnp.int32)
    be = jnp.sum((bidx[:, None] >= nb_cum[None, :]).astype(jnp.int32), axis=1)
    last_be = jnp.clip(
        jnp.sum((total_nb - 1 >= nb_cum).astype(jnp.int32)), 0, E - 1)
    be = jnp.where(bidx < total_nb, jnp.clip(be, 0, E - 1), last_be)
    act = (bidx < total_nb).astype(jnp.int32)
    return be.astype(jnp.int32), act


def _dispatch_body(x_hbm, pos_hbm, ww_hbm, xs_hbm, sw_hbm,
                   pos0_v, pos1_v, rows_v, w0_v, w1_v, sem):
    c = lax.axis_index("c")
    s = lax.axis_index("s")
    wid = s * _NC + c
    tbase = wid * _TPW
    pltpu.sync_copy(pos_hbm.at[pl.ds(tbase, _TPW)], pos0_v)
    pltpu.sync_copy(pos_hbm.at[pl.ds(T + tbase, _TPW)], pos1_v)
    pltpu.sync_copy(ww_hbm.at[pl.ds(tbase, _TPW)], w0_v)
    pltpu.sync_copy(ww_hbm.at[pl.ds(T + tbase, _TPW)], w1_v)
    pltpu.async_copy(x_hbm.at[pl.ds(tbase, _TPW)], rows_v, sem).wait()
    pltpu.sync_copy(rows_v, xs_hbm.at[pos0_v])
    pltpu.sync_copy(rows_v, xs_hbm.at[pos1_v])
    pltpu.sync_copy(w0_v, sw_hbm.at[pos0_v])
    pltpu.sync_copy(w1_v, sw_hbm.at[pos1_v])


def _dispatch(x2, pos, ww):
    mesh = plsc.VectorSubcoreMesh(core_axis_name="c", subcore_axis_name="s")
    f = pl.kernel(
        _dispatch_body,
        mesh=mesh,
        out_type=(jax.ShapeDtypeStruct((RS, D), jnp.float32),
                  jax.ShapeDtypeStruct((RS, 128), jnp.float32)),
        scratch_types=[
            pltpu.VMEM((_TPW,), jnp.int32),
            pltpu.VMEM((_TPW,), jnp.int32),
            pltpu.VMEM((_TPW, D), jnp.float32),
            pltpu.VMEM((_TPW, 128), jnp.float32),
            pltpu.VMEM((_TPW, 128), jnp.float32),
            pltpu.SemaphoreType.DMA,
        ],
    )
    return f(x2, pos, ww)


def _erf(z):
    return lax.erf(z)


def _gelu_exact(h):
    return 0.5 * h * (1.0 + _erf(h * 0.7071067811865476))


def _ffn_kernel(be_ref, act_ref, xs_ref, sw_ref, w1_ref, w2_ref, y_ref,
                w1b_ref, w2b_ref):
    b = pl.program_id(0)

    @pl.when(act_ref[b] == 1)
    def _():
        prev = be_ref[jnp.maximum(b - 1, 0)]

        @pl.when((b == 0) | (be_ref[b] != prev))
        def _cast():
            w1b_ref[...] = w1_ref[0].astype(jnp.bfloat16)
            w2b_ref[...] = w2_ref[0].astype(jnp.bfloat16)

        xb = xs_ref[...].astype(jnp.bfloat16)
        h = lax.dot_general(
            xb, w1b_ref[...], (((1,), (1,)), ((), ())),
            preferred_element_type=jnp.float32)
        a = _gelu_exact(h).astype(jnp.bfloat16)
        y = lax.dot_general(
            a, w2b_ref[...], (((1,), (1,)), ((), ())),
            preferred_element_type=jnp.float32)
        y_ref[...] = y * sw_ref[:, 0:1]


def _ffn(xs, sw, W1, W2, be, act):
    grid_spec = pltpu.PrefetchScalarGridSpec(
        num_scalar_prefetch=2,
        grid=(NB,),
        in_specs=[
            pl.BlockSpec((M, D), lambda b, be, act: (b, 0)),
            pl.BlockSpec((M, 128), lambda b, be, act: (b, 0)),
            pl.BlockSpec((1, H, D), lambda b, be, act: (be[b], 0, 0)),
            pl.BlockSpec((1, D, H), lambda b, be, act: (be[b], 0, 0)),
        ],
        out_specs=pl.BlockSpec((M, D), lambda b, be, act: (b, 0)),
        scratch_shapes=[pltpu.VMEM((H, D), jnp.bfloat16),
                        pltpu.VMEM((D, H), jnp.bfloat16)],
    )
    return pl.pallas_call(
        _ffn_kernel,
        grid_spec=grid_spec,
        out_shape=jax.ShapeDtypeStruct((RS, D), jnp.float32),
    )(be, act, xs, sw, W1, W2)


def _combine_body(y_hbm, pos_hbm, out_hbm,
                  posa0_v, posa1_v, posb0_v, posb1_v,
                  rowsa0_v, rowsa1_v, rowsb0_v, rowsb1_v,
                  acc_v, sema, semb):
    c = lax.axis_index("c")
    s = lax.axis_index("s")
    wid = s * _NC + c
    tbase = wid * _TPW
    pltpu.sync_copy(pos_hbm.at[pl.ds(tbase, _TT)], posa0_v)
    pltpu.sync_copy(pos_hbm.at[pl.ds(T + tbase, _TT)], posa1_v)
    pltpu.sync_copy(pos_hbm.at[pl.ds(tbase + _TT, _TT)], posb0_v)
    pltpu.sync_copy(pos_hbm.at[pl.ds(T + tbase + _TT, _TT)], posb1_v)
    cpa0 = pltpu.async_copy(y_hbm.at[posa0_v], rowsa0_v, sema)
    cpa1 = pltpu.async_copy(y_hbm.at[posa1_v], rowsa1_v, sema)
    cpb0 = pltpu.async_copy(y_hbm.at[posb0_v], rowsb0_v, semb)
    cpb1 = pltpu.async_copy(y_hbm.at[posb1_v], rowsb1_v, semb)

    def mk_body(r0_v, r1_v):
        def body(t, _):
            for j in range(D // 16):
                sl = pl.ds(j * 16, 16)
                acc_v[t, sl] = r0_v[t, sl] + r1_v[t, sl]
            return 0
        return body

    cpa0.wait()
    cpa1.wait()
    lax.fori_loop(0, _TT, mk_body(rowsa0_v, rowsa1_v), 0)
    pltpu.sync_copy(acc_v, out_hbm.at[pl.ds(tbase, _TT)])
    cpb0.wait()
    cpb1.wait()
    lax.fori_loop(0, _TT, mk_body(rowsb0_v, rowsb1_v), 0)
    pltpu.sync_copy(acc_v, out_hbm.at[pl.ds(tbase + _TT, _TT)])


def _combine(y, pos):
    mesh = plsc.VectorSubcoreMesh(core_axis_name="c", subcore_axis_name="s")
    f = pl.kernel(
        _combine_body,
        mesh=mesh,
        out_type=jax.ShapeDtypeStruct((T, D), jnp.float32),
        scratch_types=[
            pltpu.VMEM((_TT,), jnp.int32),
            pltpu.VMEM((_TT,), jnp.int32),
            pltpu.VMEM((_TT,), jnp.int32),
            pltpu.VMEM((_TT,), jnp.int32),
            pltpu.VMEM((_TT, D), jnp.float32),
            pltpu.VMEM((_TT, D), jnp.float32),
            pltpu.VMEM((_TT, D), jnp.float32),
            pltpu.VMEM((_TT, D), jnp.float32),
            pltpu.VMEM((_TT, D), jnp.float32),
            pltpu.SemaphoreType.DMA,
            pltpu.SemaphoreType.DMA,
        ],
    )
    return f(y, pos)


@jax.jit
def kernel(x, gate_w, W1, W2):
    b, s, d = x.shape
    x2 = x.reshape(s, d)
    logits, pos2, w, nbm = _gate(x2, gate_w)
    pos = pos2.T.reshape(P)
    be, act = _blocktable(nbm[0])
    ww = jnp.broadcast_to(w.T.reshape(P, 1), (P, 128))
    xs, sw = _dispatch(x2, pos, ww)
    y = _ffn(xs, sw, W1, W2, be, act)
    out = _combine(y, pos)
    return out.reshape(b, s, d), logits.reshape(b, s, E)


# speedup vs baseline: 1.1137x; 1.0238x over previous
import functools

import jax
import jax.numpy as jnp
from jax import lax
from jax.experimental import pallas as pl
from jax.experimental.pallas import tpu as pltpu
from jax.experimental.pallas import tpu_sc as plsc

T, D, E, H = 2048, 768, 8, 3072
K = 2
P = T * K
M = 512
NB = P // M + E
RS = NB * M
_NEG = jnp.finfo(jnp.float32).min

_NC, _NS = 2, 16
_NW = _NC * _NS
_PPW = P // _NW
_TPW = T // _NW
_TT = 32


def _gate_kernel(x_ref, gw_ref, logits_ref, pos_ref, w_ref, nb_ref):
    x = x_ref[...].astype(jnp.bfloat16)
    gw = gw_ref[...].astype(jnp.bfloat16)
    logits = lax.dot_general(
        x, gw, (((1,), (1,)), ((), ())),
        preferred_element_type=jnp.float32)
    logits_ref[...] = logits
    eidx = lax.broadcasted_iota(jnp.int32, (T, E), 1)
    m1 = jnp.max(logits, axis=1, keepdims=True)
    i1 = jnp.min(jnp.where(logits == m1, eidx, E), axis=1, keepdims=True)
    masked = jnp.where(eidx == i1, _NEG, logits)
    m2 = jnp.max(masked, axis=1, keepdims=True)
    i2 = jnp.min(jnp.where(masked == m2, eidx, E), axis=1, keepdims=True)
    ex = jnp.exp(m2 - m1)
    denom = 1.0 + ex
    w_ref[...] = jnp.concatenate([1.0 / denom, ex / denom], axis=1)

    oh1 = (eidx == i1).astype(jnp.bfloat16)
    oh2 = (eidx == i2).astype(jnp.bfloat16)
    ohb = jnp.concatenate([oh1, oh2], axis=1)
    rI = lax.broadcasted_iota(jnp.int32, (T, T), 0)
    cI = lax.broadcasted_iota(jnp.int32, (T, T), 1)
    ltri = (cI <= rI).astype(jnp.bfloat16)
    cum = lax.dot_general(
        ltri, ohb, (((1,), (0,)), ((), ())),
        preferred_element_type=jnp.float32)
    c0 = cum[T - 1:T, 0:E]
    counts = c0 + cum[T - 1:T, E:2 * E]
    nb = jnp.floor((counts + (M - 1)) * (1.0 / M))
    nb_cum = lax.dot_general(
        nb.astype(jnp.bfloat16),
        (lax.broadcasted_iota(jnp.int32, (E, E), 0)
         <= lax.broadcasted_iota(jnp.int32, (E, E), 1)).astype(jnp.bfloat16),
        (((1,), (0,)), ((), ())),
        preferred_element_type=jnp.float32)
    pbase = (nb_cum - nb) * float(M)
    o1f = oh1.astype(jnp.float32)
    o2f = oh2.astype(jnp.float32)
    pos0 = jnp.sum(o1f * (cum[:, 0:E] + pbase), axis=1, keepdims=True) - 1.0
    pos1 = jnp.sum(o2f * (cum[:, E:2 * E] + c0 + pbase),
                   axis=1, keepdims=True) - 1.0
    pos_ref[...] = jnp.concatenate([pos0, pos1], axis=1).astype(jnp.int32)
    nb_ref[...] = jnp.broadcast_to(nb, (E, E)).astype(jnp.int32)


def _gate(x2, gate_w):
    return pl.pallas_call(
        _gate_kernel,
        out_shape=(jax.ShapeDtypeStruct((T, E), jnp.float32),
                   jax.ShapeDtypeStruct((T, K), jnp.int32),
                   jax.ShapeDtypeStruct((T, K), jnp.float32),
                   jax.ShapeDtypeStruct((E, E), jnp.int32)),
    )(x2, gate_w)


def _blocktable(nbrow):
    nb_cum = jnp.cumsum(nbrow)
    total_nb = nb_cum[E - 1]
    bidx = jnp.arange(NB, dtype=jnp.int32)
    be = jnp.sum((bidx[:, None] >= nb_cum[None, :]).astype(jnp.int32), axis=1)
    last_be = jnp.clip(
        jnp.sum((total_nb - 1 >= nb_cum).astype(jnp.int32)), 0, E - 1)
    be = jnp.where(bidx < total_nb, jnp.clip(be, 0, E - 1), last_be)
    act = (bidx < total_nb).astype(jnp.int32)
    return be.astype(jnp.int32), act


def _dispatch_body(x_hbm, pos_hbm, ww_hbm, xs_hbm, sw_hbm,
                   pos0_v, pos1_v, rows_v, w0_v, w1_v, sem):
    c = lax.axis_index("c")
    s = lax.axis_index("s")
    wid = s * _NC + c
    tbase = wid * _TPW
    pltpu.sync_copy(pos_hbm.at[pl.ds(tbase, _TPW)], pos0_v)
    pltpu.sync_copy(pos_hbm.at[pl.ds(T + tbase, _TPW)], pos1_v)
    pltpu.sync_copy(ww_hbm.at[pl.ds(tbase, _TPW)], w0_v)
    pltpu.sync_copy(ww_hbm.at[pl.ds(T + tbase, _TPW)], w1_v)
    pltpu.async_copy(x_hbm.at[pl.ds(tbase, _TPW)], rows_v, sem).wait()
    pltpu.sync_copy(rows_v, xs_hbm.at[pos0_v])
    pltpu.sync_copy(rows_v, xs_hbm.at[pos1_v])
    pltpu.sync_copy(w0_v, sw_hbm.at[pos0_v])
    pltpu.sync_copy(w1_v, sw_hbm.at[pos1_v])


def _dispatch(x2, pos, ww):
    mesh = plsc.VectorSubcoreMesh(core_axis_name="c", subcore_axis_name="s")
    f = pl.kernel(
        _dispatch_body,
        mesh=mesh,
        out_type=(jax.ShapeDtypeStruct((RS, D), jnp.float32),
                  jax.ShapeDtypeStruct((RS, 128), jnp.float32)),
        scratch_types=[
            pltpu.VMEM((_TPW,), jnp.int32),
            pltpu.VMEM((_TPW,), jnp.int32),
            pltpu.VMEM((_TPW, D), jnp.float32),
            pltpu.VMEM((_TPW, 128), jnp.float32),
            pltpu.VMEM((_TPW, 128), jnp.float32),
            pltpu.SemaphoreType.DMA,
        ],
    )
    return f(x2, pos, ww)


def _erf(z):
    return lax.erf(z)


def _gelu_exact(h):
    return 0.5 * h * (1.0 + _erf(h * 0.7071067811865476))


def _ffn_kernel(be_ref, act_ref, xs_ref, sw_ref, w1_ref, w2_ref, y_ref,
                w1b_ref, w2b_ref):
    b = pl.program_id(0)

    @pl.when(act_ref[b] == 1)
    def _():
        prev = be_ref[jnp.maximum(b - 1, 0)]

        @pl.when((b == 0) | (be_ref[b] != prev))
        def _cast():
            w1b_ref[...] = w1_ref[0].astype(jnp.bfloat16)
            w2b_ref[...] = w2_ref[0].astype(jnp.bfloat16)

        xb = xs_ref[...].astype(jnp.bfloat16)
        h = lax.dot_general(
            xb, w1b_ref[...], (((1,), (1,)), ((), ())),
            preferred_element_type=jnp.float32)
        a = _gelu_exact(h).astype(jnp.bfloat16)
        y = lax.dot_general(
            a, w2b_ref[...], (((1,), (1,)), ((), ())),
            preferred_element_type=jnp.float32)
        y_ref[...] = y * sw_ref[:, 0:1]


def _ffn(xs, sw, W1, W2, be, act):
    grid_spec = pltpu.PrefetchScalarGridSpec(
        num_scalar_prefetch=2,
        grid=(NB,),
        in_specs=[
            pl.BlockSpec(
                (M, D),
                lambda b, be, act: (jnp.where(act[b] == 1, b, NB - 1), 0)),
            pl.BlockSpec(
                (M, 128),
                lambda b, be, act: (jnp.where(act[b] == 1, b, NB - 1), 0)),
            pl.BlockSpec((1, H, D), lambda b, be, act: (be[b], 0, 0)),
            pl.BlockSpec((1, D, H), lambda b, be, act: (be[b], 0, 0)),
        ],
        out_specs=pl.BlockSpec(
            (M, D),
            lambda b, be, act: (jnp.where(act[b] == 1, b, NB - 1), 0)),
        scratch_shapes=[pltpu.VMEM((H, D), jnp.bfloat16),
                        pltpu.VMEM((D, H), jnp.bfloat16)],
    )
    return pl.pallas_call(
        _ffn_kernel,
        grid_spec=grid_spec,
        out_shape=jax.ShapeDtypeStruct((RS, D), jnp.float32),
    )(be, act, xs, sw, W1, W2)


def _combine_body(y_hbm, pos_hbm, out_hbm,
                  posa0_v, posa1_v, posb0_v, posb1_v,
                  rowsa0_v, rowsa1_v, rowsb0_v, rowsb1_v,
                  acc_v, sema, semb):
    c = lax.axis_index("c")
    s = lax.axis_index("s")
    wid = s * _NC + c
    tbase = wid * _TPW
    pltpu.sync_copy(pos_hbm.at[pl.ds(tbase, _TT)], posa0_v)
    pltpu.sync_copy(pos_hbm.at[pl.ds(T + tbase, _TT)], posa1_v)
    pltpu.sync_copy(pos_hbm.at[pl.ds(tbase + _TT, _TT)], posb0_v)
    pltpu.sync_copy(pos_hbm.at[pl.ds(T + tbase + _TT, _TT)], posb1_v)
    cpa0 = pltpu.async_copy(y_hbm.at[posa0_v], rowsa0_v, sema)
    cpa1 = pltpu.async_copy(y_hbm.at[posa1_v], rowsa1_v, sema)
    cpb0 = pltpu.async_copy(y_hbm.at[posb0_v], rowsb0_v, semb)
    cpb1 = pltpu.async_copy(y_hbm.at[posb1_v], rowsb1_v, semb)

    def mk_body(r0_v, r1_v):
        def body(t, _):
            for j in range(D // 16):
                sl = pl.ds(j * 16, 16)
                acc_v[t, sl] = r0_v[t, sl] + r1_v[t, sl]
            return 0
        return body

    cpa0.wait()
    cpa1.wait()
    lax.fori_loop(0, _TT, mk_body(rowsa0_v, rowsa1_v), 0)
    pltpu.sync_copy(acc_v, out_hbm.at[pl.ds(tbase, _TT)])
    cpb0.wait()
    cpb1.wait()
    lax.fori_loop(0, _TT, mk_body(rowsb0_v, rowsb1_v), 0)
    pltpu.sync_copy(acc_v, out_hbm.at[pl.ds(tbase + _TT, _TT)])


def _combine(y, pos):
    mesh = plsc.VectorSubcoreMesh(core_axis_name="c", subcore_axis_name="s")
    f = pl.kernel(
        _combine_body,
        mesh=mesh,
        out_type=jax.ShapeDtypeStruct((T, D), jnp.float32),
        scratch_types=[
            pltpu.VMEM((_TT,), jnp.int32),
            pltpu.VMEM((_TT,), jnp.int32),
            pltpu.VMEM((_TT,), jnp.int32),
            pltpu.VMEM((_TT,), jnp.int32),
            pltpu.VMEM((_TT, D), jnp.float32),
            pltpu.VMEM((_TT, D), jnp.float32),
            pltpu.VMEM((_TT, D), jnp.float32),
            pltpu.VMEM((_TT, D), jnp.float32),
            pltpu.VMEM((_TT, D), jnp.float32),
            pltpu.SemaphoreType.DMA,
            pltpu.SemaphoreType.DMA,
        ],
    )
    return f(y, pos)


@jax.jit
def kernel(x, gate_w, W1, W2):
    b, s, d = x.shape
    x2 = x.reshape(s, d)
    logits, pos2, w, nbm = _gate(x2, gate_w)
    pos = pos2.T.reshape(P)
    be, act = _blocktable(nbm[0])
    ww = jnp.broadcast_to(w.T.reshape(P, 1), (P, 128))
    xs, sw = _dispatch(x2, pos, ww)
    y = _ffn(xs, sw, W1, W2, be, act)
    out = _combine(y, pos)
    return out.reshape(b, s, d), logits.reshape(b, s, E)
